# baseline (speedup 1.0000x reference)
import functools

import jax
import jax.numpy as jnp
from jax import lax
from jax.experimental import pallas as pl
from jax.experimental.pallas import tpu as pltpu
from jax.experimental.pallas import tpu_sc as plsc

EMB = 32
BATCH = 16384

NC = 2
NS = 16
L = 16
NW = NC * NS
BPW = BATCH // NW
GROUPS = BPW // L


def _body(user_hbm, item_hbm, uid_hbm, iid_hbm, out_hbm,
          uidx_v, iidx_v, urows_v, irows_v, hbuf_v, outv_v, sem_u, sem_i):
    wid = lax.axis_index("s") * NC + lax.axis_index("c")
    base = wid * BPW

    pltpu.sync_copy(uid_hbm.at[pl.ds(base, BPW)], uidx_v)
    pltpu.sync_copy(iid_hbm.at[pl.ds(base, BPW)], iidx_v)
    cu = pltpu.async_copy(user_hbm.at[uidx_v], urows_v, sem_u)
    ci = pltpu.async_copy(item_hbm.at[iidx_v], irows_v, sem_i)
    cu.wait()
    ci.wait()

    rows16 = lax.iota(jnp.int32, L)

    def group(g, carry):
        r0 = g * L
        for j in range(L):
            r = r0 + j
            h = (urows_v[r, pl.ds(0, L)] * irows_v[r, pl.ds(0, L)]
                 + urows_v[r, pl.ds(L, L)] * irows_v[r, pl.ds(L, L)])
            hbuf_v[j, :] = h
        cols = [plsc.load_gather(hbuf_v,
                                 [rows16, jnp.full((L,), l, jnp.int32)])
                for l in range(L)]
        while len(cols) > 1:
            cols = [cols[i] + cols[i + 1] for i in range(0, len(cols), 2)]
        outv_v[pl.ds(r0, L)] = cols[0]
        return carry

    lax.fori_loop(0, GROUPS, group, 0)
    pltpu.sync_copy(outv_v, out_hbm.at[pl.ds(base, BPW)])


@jax.jit
def kernel(user_table, item_table, u_id, i_id):
    mesh = plsc.VectorSubcoreMesh(core_axis_name="c", subcore_axis_name="s",
                                  num_cores=NC, num_subcores=NS)
    k = functools.partial(
        pl.kernel,
        out_type=jax.ShapeDtypeStruct((BATCH,), jnp.float32),
        mesh=mesh,
        scratch_types=[
            pltpu.VMEM((BPW,), jnp.int32),
            pltpu.VMEM((BPW,), jnp.int32),
            pltpu.VMEM((BPW, EMB), jnp.float32),
            pltpu.VMEM((BPW, EMB), jnp.float32),
            pltpu.VMEM((L, L), jnp.float32),
            pltpu.VMEM((BPW,), jnp.float32),
            pltpu.SemaphoreType.DMA,
            pltpu.SemaphoreType.DMA,
        ],
        compiler_params=pltpu.CompilerParams(needs_layout_passes=False,
                                             use_tc_tiling_on_sc=False),
    )(_body)
    return k(user_table, item_table,
             u_id.astype(jnp.int32), i_id.astype(jnp.int32))
